# Initial kernel scaffold; baseline (speedup 1.0000x reference)
import jax
import jax.numpy as jnp
from jax.experimental import pallas as pl


def kernel(x, edge_index, batch_idx, W1, b1, g1, be1, W2, b2, g2, be2, W3, b3, g3, be3, Wm1, bm1, Wm2, bm2, Wm3, bm3):
    raise NotImplementedError("write your pallas kernel here")


# speedup vs baseline: 12.4297x; 12.4297x over previous
import functools
import numpy as np
import jax
import jax.numpy as jnp
from jax import lax
from jax.experimental import pallas as pl
from jax.experimental.pallas import tpu as pltpu
from jax.experimental.pallas import tpu_sc as plsc

N = 50000
F = 64
HF = 32
G = 64
EPS = 1e-5
EB = 128
NBLK = 6400
EPAD = NBLK * EB
SLAB = 40
ACC_ROWS = 50048
ZROWS = ACC_ROWS // 16
OROWS = 3128
OLAST = N - 15 * OROWS
DEGW = 16
BT = 2000
GRID = N // BT


def _sc_agg_body(ytab, srcidx, dstidx, zeros_blk, out, acc, sslab, dslab, rows, sem):
    c = lax.axis_index("c")
    s = lax.axis_index("s")
    pltpu.sync_copy(zeros_blk, acc.at[pl.ds(s * ZROWS, ZROWS)])
    plsc.subcore_barrier()

    blocks_per_tile = NBLK // 16
    nslab = blocks_per_tile // SLAB

    def slab_body(k, _):
        row0 = s * blocks_per_tile + k * SLAB
        pltpu.sync_copy(srcidx.at[pl.ds(c * NBLK + row0, SLAB)], sslab)
        pltpu.sync_copy(dstidx.at[pl.ds(row0, SLAB)], dslab)

        def inner(j, _):
            pltpu.async_copy(ytab.at[sslab.at[j]], rows, sem).wait()
            pltpu.sync_copy(rows, acc.at[dslab.at[j]], add=True)
            return 0

        lax.fori_loop(0, SLAB, inner, 0)
        return 0

    lax.fori_loop(0, nslab, slab_body, 0)
    plsc.subcore_barrier()
    _writeout(acc, out, c, s)


def _writeout(acc, out, c, s):
    @pl.when(s < 15)
    def _():
        pltpu.sync_copy(
            acc.at[pl.ds(s * OROWS, OROWS)],
            out.at[pl.ds(c * N + s * OROWS, OROWS)],
        )

    @pl.when(s == 15)
    def _():
        pltpu.sync_copy(
            acc.at[pl.ds(15 * OROWS, OLAST)],
            out.at[pl.ds(c * N + 15 * OROWS, OLAST)],
        )


def _sc_deg_body(dstidx, ones_hbm, zeros_blk, out, acc, dslab, ones_v):
    c = lax.axis_index("c")
    s = lax.axis_index("s")
    pltpu.sync_copy(zeros_blk, acc.at[pl.ds(s * ZROWS, ZROWS)])
    pltpu.sync_copy(ones_hbm, ones_v)
    plsc.subcore_barrier()

    w = c * 16 + s
    blocks_per_tile = NBLK // 32
    nslab = blocks_per_tile // SLAB

    def slab_body(k, _):
        row0 = w * blocks_per_tile + k * SLAB
        pltpu.sync_copy(dstidx.at[pl.ds(row0, SLAB)], dslab)

        def inner(j, _):
            pltpu.sync_copy(ones_v, acc.at[dslab.at[j]], add=True)
            return 0

        lax.fori_loop(0, SLAB, inner, 0)
        return 0

    lax.fori_loop(0, nslab, slab_body, 0)
    plsc.subcore_barrier()
    _writeout(acc, out, c, s)


@functools.cache
def _sc_kernels():
    mesh = plsc.VectorSubcoreMesh(core_axis_name="c", subcore_axis_name="s")
    agg = functools.partial(
        pl.kernel,
        out_type=jax.ShapeDtypeStruct((2 * N, HF), jnp.float32),
        mesh=mesh,
        scratch_types=[
            pltpu.VMEM_SHARED((ACC_ROWS, HF), jnp.float32),
            pltpu.VMEM((SLAB, EB), jnp.int32),
            pltpu.VMEM((SLAB, EB), jnp.int32),
            pltpu.VMEM((EB, HF), jnp.float32),
            pltpu.SemaphoreType.DMA,
        ],
        compiler_params=pltpu.CompilerParams(use_tc_tiling_on_sc=False),
    )(_sc_agg_body)
    deg = functools.partial(
        pl.kernel,
        out_type=jax.ShapeDtypeStruct((2 * N, DEGW), jnp.float32),
        mesh=mesh,
        scratch_types=[
            pltpu.VMEM_SHARED((ACC_ROWS, DEGW), jnp.float32),
            pltpu.VMEM((SLAB, EB), jnp.int32),
            pltpu.VMEM((EB, DEGW), jnp.float32),
        ],
        compiler_params=pltpu.CompilerParams(use_tc_tiling_on_sc=False),
    )(_sc_deg_body)
    return deg, agg


def _sc_deg(dstidx, ones_hbm, zeros_blk):
    return _sc_kernels()[0](dstidx, ones_hbm, zeros_blk)


def _sc_agg(ytab, srcidx, dstidx, zeros_blk):
    return _sc_kernels()[1](ytab, srcidx, dstidx, zeros_blk)


def _t1_body(x_ref, dp_ref, w_ref, y_ref, dinv_ref):
    deg = dp_ref[0, :, 0:1] + dp_ref[1, :, 0:1] + 1.0
    dinv = lax.rsqrt(deg)
    xb = x_ref[...]
    w = w_ref[...]
    xw = (xb[:, 0:1] * w[0:1, :] + xb[:, 1:2] * w[1:2, :]
          + xb[:, 2:3] * w[2:3, :])
    yv = xw * dinv
    y_ref[0] = yv[:, :HF]
    y_ref[1] = yv[:, HF:]
    dinv_ref[...] = dinv


def _tc_first(x, degp, w1):
    return pl.pallas_call(
        _t1_body,
        grid=(GRID,),
        in_specs=[
            pl.BlockSpec((BT, 3), lambda i: (i, 0)),
            pl.BlockSpec((2, BT, DEGW), lambda i: (0, i, 0)),
            pl.BlockSpec((3, F), lambda i: (0, 0)),
        ],
        out_specs=[
            pl.BlockSpec((2, BT, HF), lambda i: (0, i, 0)),
            pl.BlockSpec((BT, 1), lambda i: (i, 0)),
        ],
        out_shape=[
            jax.ShapeDtypeStruct((2, N, HF), jnp.float32),
            jax.ShapeDtypeStruct((N, 1), jnp.float32),
        ],
    )(x, degp, w1)


def _combine(acc_ref, y_ref, dinv_ref, b_ref, g_ref, be_ref):
    agg = jnp.concatenate(
        [acc_ref[0] + y_ref[0], acc_ref[1] + y_ref[1]], axis=1)
    pre = agg * dinv_ref[...] + b_ref[...]
    scale = g_ref[...] * lax.rsqrt(jnp.float32(1.0 + EPS))
    return jnp.maximum(pre * scale + be_ref[...], 0.0)


def _t2_body(acc_ref, y_ref, dinv_ref, b_ref, g_ref, be_ref, w_ref, yout_ref):
    h = _combine(acc_ref, y_ref, dinv_ref, b_ref, g_ref, be_ref)
    ynew = jnp.dot(h, w_ref[...], preferred_element_type=jnp.float32)
    ynew = ynew * dinv_ref[...]
    yout_ref[0] = ynew[:, :HF]
    yout_ref[1] = ynew[:, HF:]


def _tc_mid(acc, y, dinv, b, g, be, wnext):
    return pl.pallas_call(
        _t2_body,
        grid=(GRID,),
        in_specs=[
            pl.BlockSpec((2, BT, HF), lambda i: (0, i, 0)),
            pl.BlockSpec((2, BT, HF), lambda i: (0, i, 0)),
            pl.BlockSpec((BT, 1), lambda i: (i, 0)),
            pl.BlockSpec((1, F), lambda i: (0, 0)),
            pl.BlockSpec((1, F), lambda i: (0, 0)),
            pl.BlockSpec((1, F), lambda i: (0, 0)),
            pl.BlockSpec((F, F), lambda i: (0, 0)),
        ],
        out_specs=pl.BlockSpec((2, BT, HF), lambda i: (0, i, 0)),
        out_shape=jax.ShapeDtypeStruct((2, N, HF), jnp.float32),
    )(acc, y, dinv, b, g, be, wnext)


def _t4_body(acc_ref, y_ref, dinv_ref, b_ref, g_ref, be_ref, batch_ref,
             wm1_ref, bm1_ref, wm2_ref, bm2_ref, wm3_ref, bm3_ref,
             out_ref, sum_s, cnt_s, max_s):
    @pl.when(pl.program_id(0) == 0)
    def _():
        sum_s[...] = jnp.zeros_like(sum_s)
        cnt_s[...] = jnp.zeros_like(cnt_s)
        max_s[...] = jnp.full_like(max_s, -jnp.inf)

    h = _combine(acc_ref, y_ref, dinv_ref, b_ref, g_ref, be_ref)
    batch = batch_ref[...]
    oh = (batch == lax.broadcasted_iota(jnp.int32, (BT, G), 1))
    oh = oh.astype(jnp.float32)
    dn = (((0,), (0,)), ((), ()))
    sum_s[...] += lax.dot_general(oh, h, dn,
                                  preferred_element_type=jnp.float32)
    cnt_s[...] += lax.dot_general(oh, jnp.ones((BT, 1), jnp.float32), dn,
                                  preferred_element_type=jnp.float32)

    glo = jnp.min(batch)
    ghi = jnp.max(batch)
    rowid = lax.broadcasted_iota(jnp.int32, (G, F), 0)

    def mbody(g, _):
        m = jnp.max(jnp.where(batch == g, h, -jnp.inf), axis=0, keepdims=True)
        upd = jnp.where(rowid == g, jnp.broadcast_to(m, (G, F)), -jnp.inf)
        max_s[...] = jnp.maximum(max_s[...], upd)
        return 0

    lax.fori_loop(glo, ghi + 1, mbody, 0)

    @pl.when(pl.program_id(0) == GRID - 1)
    def _():
        mean = sum_s[...] / jnp.maximum(cnt_s[...], 1.0)
        xg = jnp.concatenate([mean, max_s[...]], axis=1)
        z = jnp.maximum(jnp.dot(xg, wm1_ref[...],
                                preferred_element_type=jnp.float32)
                        + bm1_ref[...], 0.0)
        z = jnp.maximum(jnp.dot(z, wm2_ref[...],
                                preferred_element_type=jnp.float32)
                        + bm2_ref[...], 0.0)
        out_ref[...] = jnp.dot(z, wm3_ref[...],
                               preferred_element_type=jnp.float32) + bm3_ref[...]


def _tc_final(acc, y, dinv, b, g, be, batch2d, wm1, bm1, wm2, bm2, wm3, bm3):
    full = lambda r, c: pl.BlockSpec((r, c), lambda i: (0, 0))
    return pl.pallas_call(
        _t4_body,
        grid=(GRID,),
        in_specs=[
            pl.BlockSpec((2, BT, HF), lambda i: (0, i, 0)),
            pl.BlockSpec((2, BT, HF), lambda i: (0, i, 0)),
            pl.BlockSpec((BT, 1), lambda i: (i, 0)),
            full(1, F), full(1, F), full(1, F),
            pl.BlockSpec((BT, 1), lambda i: (i, 0)),
            full(2 * F, 32), full(1, 32),
            full(32, 16), full(1, 16),
            full(16, 1), full(1, 1),
        ],
        out_specs=pl.BlockSpec((G, 1), lambda i: (0, 0)),
        out_shape=jax.ShapeDtypeStruct((G, 1), jnp.float32),
        scratch_shapes=[
            pltpu.VMEM((G, F), jnp.float32),
            pltpu.VMEM((G, 1), jnp.float32),
            pltpu.VMEM((G, F), jnp.float32),
        ],
    )(acc, y, dinv, b, g, be, batch2d, wm1, bm1, wm2, bm2, wm3, bm3)


def kernel(x, edge_index, batch_idx, W1, b1, g1, be1, W2, b2, g2, be2,
           W3, b3, g3, be3, Wm1, bm1, Wm2, bm2, Wm3, bm3):
    ei = edge_index.astype(jnp.int32)
    npad = EPAD - ei.shape[1]
    src = jnp.concatenate([ei[0], jnp.zeros((npad,), jnp.int32)])
    dst = jnp.concatenate([ei[1], jnp.full((npad,), N, jnp.int32)])
    srcidx = jnp.concatenate([src, src + N]).reshape(2 * NBLK, EB)
    dstidx = dst.reshape(NBLK, EB)

    ones_hbm = jnp.ones((EB, DEGW), jnp.float32)
    zeros16 = jnp.zeros((ZROWS, DEGW), jnp.float32)
    zeros32 = jnp.zeros((ZROWS, HF), jnp.float32)

    degp = _sc_deg(dstidx, ones_hbm, zeros16)
    degp = degp.reshape(2, N, DEGW)

    y1v, dinv = _tc_first(x, degp, W1)

    b1r, g1r, be1r = b1.reshape(1, F), g1.reshape(1, F), be1.reshape(1, F)
    b2r, g2r, be2r = b2.reshape(1, F), g2.reshape(1, F), be2.reshape(1, F)
    b3r, g3r, be3r = b3.reshape(1, F), g3.reshape(1, F), be3.reshape(1, F)

    acc1 = _sc_agg(y1v.reshape(2 * N, HF), srcidx, dstidx,
                   zeros32).reshape(2, N, HF)
    y2v = _tc_mid(acc1, y1v, dinv, b1r, g1r, be1r, W2)
    acc2 = _sc_agg(y2v.reshape(2 * N, HF), srcidx, dstidx,
                   zeros32).reshape(2, N, HF)
    y3v = _tc_mid(acc2, y2v, dinv, b2r, g2r, be2r, W3)
    acc3 = _sc_agg(y3v.reshape(2 * N, HF), srcidx, dstidx,
                   zeros32).reshape(2, N, HF)

    batch2d = batch_idx.astype(jnp.int32).reshape(N, 1)
    return _tc_final(acc3, y3v, dinv, b3r, g3r, be3r, batch2d,
                     Wm1, bm1.reshape(1, 32), Wm2, bm2.reshape(1, 16),
                     Wm3, bm3.reshape(1, 1))


# speedup vs baseline: 16.7242x; 1.3455x over previous
import functools
import numpy as np
import jax
import jax.numpy as jnp
from jax import lax
from jax.experimental import pallas as pl
from jax.experimental.pallas import tpu as pltpu
from jax.experimental.pallas import tpu_sc as plsc

N = 50000
F = 64
HF = 32
G = 64
EPS = 1e-5
EB = 128
NBLK = 6400
EPAD = NBLK * EB
SLAB = 16
DSLAB = 40
ACC_ROWS = 50048
ZROWS = ACC_ROWS // 16
OROWS = 3128
OLAST = N - 15 * OROWS
DEGW = 16
BT = 2000
GRID = N // BT


NB = 4
D = 2


def _sc_agg_body(ytab, srcidx, dstidx, zeros_blk, out, acc, sslab, dslab,
                 rows, sems_g, sems_s):
    c = lax.axis_index("c")
    s = lax.axis_index("s")
    pltpu.sync_copy(zeros_blk, acc.at[pl.ds(s * ZROWS, ZROWS)])
    plsc.subcore_barrier()

    blocks_per_tile = NBLK // 16
    nslab = blocks_per_tile // SLAB

    def slab_body(k, _):
        row0 = s * blocks_per_tile + k * SLAB
        pltpu.sync_copy(srcidx.at[pl.ds(c * NBLK + row0, SLAB)], sslab)
        pltpu.sync_copy(dstidx.at[pl.ds(row0, SLAB)], dslab)

        for b in range(D):
            pltpu.async_copy(ytab.at[sslab.at[b]], rows[b], sems_g[b])

        def inner(o, _):
            for t in range(NB):
                i = o * NB + t
                bg = (t + D) % NB

                @pl.when(i + D >= NB)
                def _():
                    pltpu.make_async_copy(
                        rows[bg], acc.at[pl.ds(0, EB)], sems_s[bg]).wait()

                @pl.when(i + D < SLAB)
                def _():
                    pltpu.async_copy(
                        ytab.at[sslab.at[i + D]], rows[bg], sems_g[bg])

                pltpu.make_async_copy(
                    ytab.at[pl.ds(0, EB)], rows[t], sems_g[t]).wait()
                pltpu.async_copy(
                    rows[t], acc.at[dslab.at[i]], sems_s[t], add=True)
            return 0

        lax.fori_loop(0, SLAB // NB, inner, 0)
        for t in range(NB - D, NB):
            pltpu.make_async_copy(
                rows[t], acc.at[pl.ds(0, EB)], sems_s[t]).wait()
        return 0

    lax.fori_loop(0, nslab, slab_body, 0)
    plsc.subcore_barrier()
    _writeout(acc, out, c, s)


def _writeout(acc, out, c, s):
    @pl.when(s < 15)
    def _():
        pltpu.sync_copy(
            acc.at[pl.ds(s * OROWS, OROWS)],
            out.at[pl.ds(c * N + s * OROWS, OROWS)],
        )

    @pl.when(s == 15)
    def _():
        pltpu.sync_copy(
            acc.at[pl.ds(15 * OROWS, OLAST)],
            out.at[pl.ds(c * N + 15 * OROWS, OLAST)],
        )


def _sc_deg_body(dstidx, ones_hbm, zeros_blk, out, acc, dslab, ones_v, sem):
    c = lax.axis_index("c")
    s = lax.axis_index("s")
    pltpu.sync_copy(zeros_blk, acc.at[pl.ds(s * ZROWS, ZROWS)])
    pltpu.sync_copy(ones_hbm, ones_v)
    plsc.subcore_barrier()

    w = c * 16 + s
    blocks_per_tile = NBLK // 32
    nslab = blocks_per_tile // DSLAB

    def slab_body(k, _):
        row0 = w * blocks_per_tile + k * DSLAB
        pltpu.sync_copy(dstidx.at[pl.ds(row0, DSLAB)], dslab)

        def fire(j, _):
            pltpu.async_copy(ones_v, acc.at[dslab.at[j]], sem, add=True)
            return 0

        lax.fori_loop(0, DSLAB, fire, 0)

        def drain(j, _):
            pltpu.make_async_copy(ones_v, acc.at[pl.ds(0, EB)], sem).wait()
            return 0

        lax.fori_loop(0, DSLAB, drain, 0)
        return 0

    lax.fori_loop(0, nslab, slab_body, 0)
    plsc.subcore_barrier()
    _writeout(acc, out, c, s)


@functools.cache
def _sc_kernels():
    mesh = plsc.VectorSubcoreMesh(core_axis_name="c", subcore_axis_name="s")
    agg = functools.partial(
        pl.kernel,
        out_type=jax.ShapeDtypeStruct((2 * N, HF), jnp.float32),
        mesh=mesh,
        scratch_types=[
            pltpu.VMEM_SHARED((ACC_ROWS, HF), jnp.float32),
            pltpu.VMEM((SLAB, EB), jnp.int32),
            pltpu.VMEM((SLAB, EB), jnp.int32),
            [pltpu.VMEM((EB, HF), jnp.float32) for _ in range(NB)],
            [pltpu.SemaphoreType.DMA for _ in range(NB)],
            [pltpu.SemaphoreType.DMA for _ in range(NB)],
        ],
        compiler_params=pltpu.CompilerParams(use_tc_tiling_on_sc=False),
    )(_sc_agg_body)
    deg = functools.partial(
        pl.kernel,
        out_type=jax.ShapeDtypeStruct((2 * N, DEGW), jnp.float32),
        mesh=mesh,
        scratch_types=[
            pltpu.VMEM_SHARED((ACC_ROWS, DEGW), jnp.float32),
            pltpu.VMEM((DSLAB, EB), jnp.int32),
            pltpu.VMEM((EB, DEGW), jnp.float32),
            pltpu.SemaphoreType.DMA,
        ],
        compiler_params=pltpu.CompilerParams(use_tc_tiling_on_sc=False),
    )(_sc_deg_body)
    return deg, agg


def _sc_deg(dstidx, ones_hbm, zeros_blk):
    return _sc_kernels()[0](dstidx, ones_hbm, zeros_blk)


def _sc_agg(ytab, srcidx, dstidx, zeros_blk):
    return _sc_kernels()[1](ytab, srcidx, dstidx, zeros_blk)


def _t1_body(x_ref, dp_ref, w_ref, y_ref, dinv_ref):
    deg = dp_ref[0, :, 0:1] + dp_ref[1, :, 0:1] + 1.0
    dinv = lax.rsqrt(deg)
    xb = x_ref[...]
    w = w_ref[...]
    xw = (xb[:, 0:1] * w[0:1, :] + xb[:, 1:2] * w[1:2, :]
          + xb[:, 2:3] * w[2:3, :])
    yv = xw * dinv
    y_ref[0] = yv[:, :HF]
    y_ref[1] = yv[:, HF:]
    dinv_ref[...] = dinv


def _tc_first(x, degp, w1):
    return pl.pallas_call(
        _t1_body,
        grid=(GRID,),
        in_specs=[
            pl.BlockSpec((BT, 3), lambda i: (i, 0)),
            pl.BlockSpec((2, BT, DEGW), lambda i: (0, i, 0)),
            pl.BlockSpec((3, F), lambda i: (0, 0)),
        ],
        out_specs=[
            pl.BlockSpec((2, BT, HF), lambda i: (0, i, 0)),
            pl.BlockSpec((BT, 1), lambda i: (i, 0)),
        ],
        out_shape=[
            jax.ShapeDtypeStruct((2, N, HF), jnp.float32),
            jax.ShapeDtypeStruct((N, 1), jnp.float32),
        ],
    )(x, degp, w1)


def _combine(acc_ref, y_ref, dinv_ref, b_ref, g_ref, be_ref):
    agg = jnp.concatenate(
        [acc_ref[0] + y_ref[0], acc_ref[1] + y_ref[1]], axis=1)
    pre = agg * dinv_ref[...] + b_ref[...]
    scale = g_ref[...] * lax.rsqrt(jnp.float32(1.0 + EPS))
    return jnp.maximum(pre * scale + be_ref[...], 0.0)


def _t2_body(acc_ref, y_ref, dinv_ref, b_ref, g_ref, be_ref, w_ref, yout_ref):
    h = _combine(acc_ref, y_ref, dinv_ref, b_ref, g_ref, be_ref)
    ynew = jnp.dot(h, w_ref[...], preferred_element_type=jnp.float32)
    ynew = ynew * dinv_ref[...]
    yout_ref[0] = ynew[:, :HF]
    yout_ref[1] = ynew[:, HF:]


def _tc_mid(acc, y, dinv, b, g, be, wnext):
    return pl.pallas_call(
        _t2_body,
        grid=(GRID,),
        in_specs=[
            pl.BlockSpec((2, BT, HF), lambda i: (0, i, 0)),
            pl.BlockSpec((2, BT, HF), lambda i: (0, i, 0)),
            pl.BlockSpec((BT, 1), lambda i: (i, 0)),
            pl.BlockSpec((1, F), lambda i: (0, 0)),
            pl.BlockSpec((1, F), lambda i: (0, 0)),
            pl.BlockSpec((1, F), lambda i: (0, 0)),
            pl.BlockSpec((F, F), lambda i: (0, 0)),
        ],
        out_specs=pl.BlockSpec((2, BT, HF), lambda i: (0, i, 0)),
        out_shape=jax.ShapeDtypeStruct((2, N, HF), jnp.float32),
    )(acc, y, dinv, b, g, be, wnext)


def _t4_body(acc_ref, y_ref, dinv_ref, b_ref, g_ref, be_ref, batch_ref,
             wm1_ref, bm1_ref, wm2_ref, bm2_ref, wm3_ref, bm3_ref,
             out_ref, sum_s, cnt_s, max_s):
    @pl.when(pl.program_id(0) == 0)
    def _():
        sum_s[...] = jnp.zeros_like(sum_s)
        cnt_s[...] = jnp.zeros_like(cnt_s)
        max_s[...] = jnp.full_like(max_s, -jnp.inf)

    h = _combine(acc_ref, y_ref, dinv_ref, b_ref, g_ref, be_ref)
    batch = batch_ref[...]
    oh = (batch == lax.broadcasted_iota(jnp.int32, (BT, G), 1))
    oh = oh.astype(jnp.float32)
    dn = (((0,), (0,)), ((), ()))
    sum_s[...] += lax.dot_general(oh, h, dn,
                                  preferred_element_type=jnp.float32)
    cnt_s[...] += lax.dot_general(oh, jnp.ones((BT, 1), jnp.float32), dn,
                                  preferred_element_type=jnp.float32)

    glo = jnp.min(batch)
    ghi = jnp.max(batch)
    rowid = lax.broadcasted_iota(jnp.int32, (G, F), 0)

    def mbody(g, _):
        m = jnp.max(jnp.where(batch == g, h, -jnp.inf), axis=0, keepdims=True)
        upd = jnp.where(rowid == g, jnp.broadcast_to(m, (G, F)), -jnp.inf)
        max_s[...] = jnp.maximum(max_s[...], upd)
        return 0

    lax.fori_loop(glo, ghi + 1, mbody, 0)

    @pl.when(pl.program_id(0) == GRID - 1)
    def _():
        mean = sum_s[...] / jnp.maximum(cnt_s[...], 1.0)
        xg = jnp.concatenate([mean, max_s[...]], axis=1)
        z = jnp.maximum(jnp.dot(xg, wm1_ref[...],
                                preferred_element_type=jnp.float32)
                        + bm1_ref[...], 0.0)
        z = jnp.maximum(jnp.dot(z, wm2_ref[...],
                                preferred_element_type=jnp.float32)
                        + bm2_ref[...], 0.0)
        out_ref[...] = jnp.dot(z, wm3_ref[...],
                               preferred_element_type=jnp.float32) + bm3_ref[...]


def _tc_final(acc, y, dinv, b, g, be, batch2d, wm1, bm1, wm2, bm2, wm3, bm3):
    full = lambda r, c: pl.BlockSpec((r, c), lambda i: (0, 0))
    return pl.pallas_call(
        _t4_body,
        grid=(GRID,),
        in_specs=[
            pl.BlockSpec((2, BT, HF), lambda i: (0, i, 0)),
            pl.BlockSpec((2, BT, HF), lambda i: (0, i, 0)),
            pl.BlockSpec((BT, 1), lambda i: (i, 0)),
            full(1, F), full(1, F), full(1, F),
            pl.BlockSpec((BT, 1), lambda i: (i, 0)),
            full(2 * F, 32), full(1, 32),
            full(32, 16), full(1, 16),
            full(16, 1), full(1, 1),
        ],
        out_specs=pl.BlockSpec((G, 1), lambda i: (0, 0)),
        out_shape=jax.ShapeDtypeStruct((G, 1), jnp.float32),
        scratch_shapes=[
            pltpu.VMEM((G, F), jnp.float32),
            pltpu.VMEM((G, 1), jnp.float32),
            pltpu.VMEM((G, F), jnp.float32),
        ],
    )(acc, y, dinv, b, g, be, batch2d, wm1, bm1, wm2, bm2, wm3, bm3)


def kernel(x, edge_index, batch_idx, W1, b1, g1, be1, W2, b2, g2, be2,
           W3, b3, g3, be3, Wm1, bm1, Wm2, bm2, Wm3, bm3):
    ei = edge_index.astype(jnp.int32)
    npad = EPAD - ei.shape[1]
    src = jnp.concatenate([ei[0], jnp.zeros((npad,), jnp.int32)])
    dst = jnp.concatenate([ei[1], jnp.full((npad,), N, jnp.int32)])
    srcidx = jnp.concatenate([src, src + N]).reshape(2 * NBLK, EB)
    dstidx = dst.reshape(NBLK, EB)

    ones_hbm = jnp.ones((EB, DEGW), jnp.float32)
    zeros16 = jnp.zeros((ZROWS, DEGW), jnp.float32)
    zeros32 = jnp.zeros((ZROWS, HF), jnp.float32)

    degp = _sc_deg(dstidx, ones_hbm, zeros16)
    degp = degp.reshape(2, N, DEGW)

    y1v, dinv = _tc_first(x, degp, W1)

    b1r, g1r, be1r = b1.reshape(1, F), g1.reshape(1, F), be1.reshape(1, F)
    b2r, g2r, be2r = b2.reshape(1, F), g2.reshape(1, F), be2.reshape(1, F)
    b3r, g3r, be3r = b3.reshape(1, F), g3.reshape(1, F), be3.reshape(1, F)

    acc1 = _sc_agg(y1v.reshape(2 * N, HF), srcidx, dstidx,
                   zeros32).reshape(2, N, HF)
    y2v = _tc_mid(acc1, y1v, dinv, b1r, g1r, be1r, W2)
    acc2 = _sc_agg(y2v.reshape(2 * N, HF), srcidx, dstidx,
                   zeros32).reshape(2, N, HF)
    y3v = _tc_mid(acc2, y2v, dinv, b2r, g2r, be2r, W3)
    acc3 = _sc_agg(y3v.reshape(2 * N, HF), srcidx, dstidx,
                   zeros32).reshape(2, N, HF)

    batch2d = batch_idx.astype(jnp.int32).reshape(N, 1)
    return _tc_final(acc3, y3v, dinv, b3r, g3r, be3r, batch2d,
                     Wm1, bm1.reshape(1, 32), Wm2, bm2.reshape(1, 16),
                     Wm3, bm3.reshape(1, 1))


# speedup vs baseline: 17.4399x; 1.0428x over previous
import functools
import numpy as np
import jax
import jax.numpy as jnp
from jax import lax
from jax.experimental import pallas as pl
from jax.experimental.pallas import tpu as pltpu
from jax.experimental.pallas import tpu_sc as plsc

N = 50000
F = 64
HF = 32
G = 64
EPS = 1e-5
EB = 128
NBLK = 6400
EPAD = NBLK * EB
SLAB = 40
DSLAB = 40
ACC_ROWS = 50048
ZROWS = ACC_ROWS // 16
OROWS = 3128
OLAST = N - 15 * OROWS
DEGW = 16
BT = 2000
GRID = N // BT


NB = 5
D = 2


def _sc_agg_body(ytab, srcidx, dstidx, zeros_blk, out, acc, sslab, dslab,
                 rows, sems_g, sems_s):
    c = lax.axis_index("c")
    s = lax.axis_index("s")
    pltpu.sync_copy(zeros_blk, acc.at[pl.ds(s * ZROWS, ZROWS)])
    plsc.subcore_barrier()

    blocks_per_tile = NBLK // 16
    nslab = blocks_per_tile // SLAB

    def slab_body(k, _):
        row0 = s * blocks_per_tile + k * SLAB
        pltpu.sync_copy(srcidx.at[pl.ds(c * NBLK + row0, SLAB)], sslab)
        pltpu.sync_copy(dstidx.at[pl.ds(row0, SLAB)], dslab)

        for b in range(D):
            pltpu.async_copy(ytab.at[sslab.at[b]], rows[b], sems_g[b])

        def inner(o, _):
            for t in range(NB):
                i = o * NB + t
                bg = (t + D) % NB

                @pl.when(i + D >= NB)
                def _():
                    pltpu.make_async_copy(
                        rows[bg], acc.at[pl.ds(0, EB)], sems_s[bg]).wait()

                @pl.when(i + D < SLAB)
                def _():
                    pltpu.async_copy(
                        ytab.at[sslab.at[i + D]], rows[bg], sems_g[bg])

                pltpu.make_async_copy(
                    ytab.at[pl.ds(0, EB)], rows[t], sems_g[t]).wait()
                pltpu.async_copy(
                    rows[t], acc.at[dslab.at[i]], sems_s[t], add=True)
            return 0

        lax.fori_loop(0, SLAB // NB, inner, 0)
        for t in range(NB - D, NB):
            pltpu.make_async_copy(
                rows[t], acc.at[pl.ds(0, EB)], sems_s[t]).wait()
        return 0

    lax.fori_loop(0, nslab, slab_body, 0)
    plsc.subcore_barrier()
    _writeout(acc, out, c, s)


def _writeout(acc, out, c, s):
    @pl.when(s < 15)
    def _():
        pltpu.sync_copy(
            acc.at[pl.ds(s * OROWS, OROWS)],
            out.at[pl.ds(c * N + s * OROWS, OROWS)],
        )

    @pl.when(s == 15)
    def _():
        pltpu.sync_copy(
            acc.at[pl.ds(15 * OROWS, OLAST)],
            out.at[pl.ds(c * N + 15 * OROWS, OLAST)],
        )


def _sc_deg_body(dstidx, ones_hbm, zeros_blk, out, acc, dslab, ones_v, sem):
    c = lax.axis_index("c")
    s = lax.axis_index("s")
    pltpu.sync_copy(zeros_blk, acc.at[pl.ds(s * ZROWS, ZROWS)])
    pltpu.sync_copy(ones_hbm, ones_v)
    plsc.subcore_barrier()

    w = c * 16 + s
    blocks_per_tile = NBLK // 32
    nslab = blocks_per_tile // DSLAB

    def slab_body(k, _):
        row0 = w * blocks_per_tile + k * DSLAB
        pltpu.sync_copy(dstidx.at[pl.ds(row0, DSLAB)], dslab)

        def fire(j, _):
            pltpu.async_copy(ones_v, acc.at[dslab.at[j]], sem, add=True)
            return 0

        lax.fori_loop(0, DSLAB, fire, 0)

        def drain(j, _):
            pltpu.make_async_copy(ones_v, acc.at[pl.ds(0, EB)], sem).wait()
            return 0

        lax.fori_loop(0, DSLAB, drain, 0)
        return 0

    lax.fori_loop(0, nslab, slab_body, 0)
    plsc.subcore_barrier()
    _writeout(acc, out, c, s)


@functools.cache
def _sc_kernels():
    mesh = plsc.VectorSubcoreMesh(core_axis_name="c", subcore_axis_name="s")
    agg = functools.partial(
        pl.kernel,
        out_type=jax.ShapeDtypeStruct((2 * N, HF), jnp.float32),
        mesh=mesh,
        scratch_types=[
            pltpu.VMEM_SHARED((ACC_ROWS, HF), jnp.float32),
            pltpu.VMEM((SLAB, EB), jnp.int32),
            pltpu.VMEM((SLAB, EB), jnp.int32),
            [pltpu.VMEM((EB, HF), jnp.float32) for _ in range(NB)],
            [pltpu.SemaphoreType.DMA for _ in range(NB)],
            [pltpu.SemaphoreType.DMA for _ in range(NB)],
        ],
        compiler_params=pltpu.CompilerParams(use_tc_tiling_on_sc=False),
    )(_sc_agg_body)
    deg = functools.partial(
        pl.kernel,
        out_type=jax.ShapeDtypeStruct((2 * N, DEGW), jnp.float32),
        mesh=mesh,
        scratch_types=[
            pltpu.VMEM_SHARED((ACC_ROWS, DEGW), jnp.float32),
            pltpu.VMEM((DSLAB, EB), jnp.int32),
            pltpu.VMEM((EB, DEGW), jnp.float32),
            pltpu.SemaphoreType.DMA,
        ],
        compiler_params=pltpu.CompilerParams(use_tc_tiling_on_sc=False),
    )(_sc_deg_body)
    return deg, agg


def _sc_deg(dstidx, ones_hbm, zeros_blk):
    return _sc_kernels()[0](dstidx, ones_hbm, zeros_blk)


def _sc_agg(ytab, srcidx, dstidx, zeros_blk):
    return _sc_kernels()[1](ytab, srcidx, dstidx, zeros_blk)


def _t1_body(x_ref, dp_ref, w_ref, y_ref, dinv_ref):
    deg = dp_ref[0, :, 0:1] + dp_ref[1, :, 0:1] + 1.0
    dinv = lax.rsqrt(deg)
    xb = x_ref[...]
    w = w_ref[...]
    xw = (xb[:, 0:1] * w[0:1, :] + xb[:, 1:2] * w[1:2, :]
          + xb[:, 2:3] * w[2:3, :])
    yv = xw * dinv
    y_ref[0] = yv[:, :HF]
    y_ref[1] = yv[:, HF:]
    dinv_ref[...] = dinv


def _tc_first(x, degp, w1):
    return pl.pallas_call(
        _t1_body,
        grid=(GRID,),
        in_specs=[
            pl.BlockSpec((BT, 3), lambda i: (i, 0)),
            pl.BlockSpec((2, BT, DEGW), lambda i: (0, i, 0)),
            pl.BlockSpec((3, F), lambda i: (0, 0)),
        ],
        out_specs=[
            pl.BlockSpec((2, BT, HF), lambda i: (0, i, 0)),
            pl.BlockSpec((BT, 1), lambda i: (i, 0)),
        ],
        out_shape=[
            jax.ShapeDtypeStruct((2, N, HF), jnp.float32),
            jax.ShapeDtypeStruct((N, 1), jnp.float32),
        ],
    )(x, degp, w1)


def _combine(acc_ref, y_ref, dinv_ref, b_ref, g_ref, be_ref):
    agg = jnp.concatenate(
        [acc_ref[0] + y_ref[0], acc_ref[1] + y_ref[1]], axis=1)
    pre = agg * dinv_ref[...] + b_ref[...]
    scale = g_ref[...] * lax.rsqrt(jnp.float32(1.0 + EPS))
    return jnp.maximum(pre * scale + be_ref[...], 0.0)


def _t2_body(acc_ref, y_ref, dinv_ref, b_ref, g_ref, be_ref, w_ref, yout_ref):
    h = _combine(acc_ref, y_ref, dinv_ref, b_ref, g_ref, be_ref)
    ynew = jnp.dot(h, w_ref[...], preferred_element_type=jnp.float32)
    ynew = ynew * dinv_ref[...]
    yout_ref[0] = ynew[:, :HF]
    yout_ref[1] = ynew[:, HF:]


def _tc_mid(acc, y, dinv, b, g, be, wnext):
    return pl.pallas_call(
        _t2_body,
        grid=(GRID,),
        in_specs=[
            pl.BlockSpec((2, BT, HF), lambda i: (0, i, 0)),
            pl.BlockSpec((2, BT, HF), lambda i: (0, i, 0)),
            pl.BlockSpec((BT, 1), lambda i: (i, 0)),
            pl.BlockSpec((1, F), lambda i: (0, 0)),
            pl.BlockSpec((1, F), lambda i: (0, 0)),
            pl.BlockSpec((1, F), lambda i: (0, 0)),
            pl.BlockSpec((F, F), lambda i: (0, 0)),
        ],
        out_specs=pl.BlockSpec((2, BT, HF), lambda i: (0, i, 0)),
        out_shape=jax.ShapeDtypeStruct((2, N, HF), jnp.float32),
    )(acc, y, dinv, b, g, be, wnext)


def _t4_body(acc_ref, y_ref, dinv_ref, b_ref, g_ref, be_ref, batch_ref,
             wm1_ref, bm1_ref, wm2_ref, bm2_ref, wm3_ref, bm3_ref,
             out_ref, sum_s, cnt_s, max_s):
    @pl.when(pl.program_id(0) == 0)
    def _():
        sum_s[...] = jnp.zeros_like(sum_s)
        cnt_s[...] = jnp.zeros_like(cnt_s)
        max_s[...] = jnp.full_like(max_s, -jnp.inf)

    h = _combine(acc_ref, y_ref, dinv_ref, b_ref, g_ref, be_ref)
    batch = batch_ref[...]
    oh = (batch == lax.broadcasted_iota(jnp.int32, (BT, G), 1))
    oh = oh.astype(jnp.float32)
    dn = (((0,), (0,)), ((), ()))
    sum_s[...] += lax.dot_general(oh, h, dn,
                                  preferred_element_type=jnp.float32)
    cnt_s[...] += lax.dot_general(oh, jnp.ones((BT, 1), jnp.float32), dn,
                                  preferred_element_type=jnp.float32)

    glo = jnp.min(batch)
    ghi = jnp.max(batch)
    rowid = lax.broadcasted_iota(jnp.int32, (G, F), 0)

    def mbody(g, _):
        m = jnp.max(jnp.where(batch == g, h, -jnp.inf), axis=0, keepdims=True)
        upd = jnp.where(rowid == g, jnp.broadcast_to(m, (G, F)), -jnp.inf)
        max_s[...] = jnp.maximum(max_s[...], upd)
        return 0

    lax.fori_loop(glo, ghi + 1, mbody, 0)

    @pl.when(pl.program_id(0) == GRID - 1)
    def _():
        mean = sum_s[...] / jnp.maximum(cnt_s[...], 1.0)
        xg = jnp.concatenate([mean, max_s[...]], axis=1)
        z = jnp.maximum(jnp.dot(xg, wm1_ref[...],
                                preferred_element_type=jnp.float32)
                        + bm1_ref[...], 0.0)
        z = jnp.maximum(jnp.dot(z, wm2_ref[...],
                                preferred_element_type=jnp.float32)
                        + bm2_ref[...], 0.0)
        out_ref[...] = jnp.dot(z, wm3_ref[...],
                               preferred_element_type=jnp.float32) + bm3_ref[...]


def _tc_final(acc, y, dinv, b, g, be, batch2d, wm1, bm1, wm2, bm2, wm3, bm3):
    full = lambda r, c: pl.BlockSpec((r, c), lambda i: (0, 0))
    return pl.pallas_call(
        _t4_body,
        grid=(GRID,),
        in_specs=[
            pl.BlockSpec((2, BT, HF), lambda i: (0, i, 0)),
            pl.BlockSpec((2, BT, HF), lambda i: (0, i, 0)),
            pl.BlockSpec((BT, 1), lambda i: (i, 0)),
            full(1, F), full(1, F), full(1, F),
            pl.BlockSpec((BT, 1), lambda i: (i, 0)),
            full(2 * F, 32), full(1, 32),
            full(32, 16), full(1, 16),
            full(16, 1), full(1, 1),
        ],
        out_specs=pl.BlockSpec((G, 1), lambda i: (0, 0)),
        out_shape=jax.ShapeDtypeStruct((G, 1), jnp.float32),
        scratch_shapes=[
            pltpu.VMEM((G, F), jnp.float32),
            pltpu.VMEM((G, 1), jnp.float32),
            pltpu.VMEM((G, F), jnp.float32),
        ],
    )(acc, y, dinv, b, g, be, batch2d, wm1, bm1, wm2, bm2, wm3, bm3)


def kernel(x, edge_index, batch_idx, W1, b1, g1, be1, W2, b2, g2, be2,
           W3, b3, g3, be3, Wm1, bm1, Wm2, bm2, Wm3, bm3):
    ei = edge_index.astype(jnp.int32)
    npad = EPAD - ei.shape[1]
    src = jnp.concatenate([ei[0], jnp.zeros((npad,), jnp.int32)])
    dst = jnp.concatenate([ei[1], jnp.full((npad,), N, jnp.int32)])
    srcidx = jnp.concatenate([src, src + N]).reshape(2 * NBLK, EB)
    dstidx = dst.reshape(NBLK, EB)

    ones_hbm = jnp.ones((EB, DEGW), jnp.float32)
    zeros16 = jnp.zeros((ZROWS, DEGW), jnp.float32)
    zeros32 = jnp.zeros((ZROWS, HF), jnp.float32)

    degp = _sc_deg(dstidx, ones_hbm, zeros16)
    degp = degp.reshape(2, N, DEGW)

    y1v, dinv = _tc_first(x, degp, W1)

    b1r, g1r, be1r = b1.reshape(1, F), g1.reshape(1, F), be1.reshape(1, F)
    b2r, g2r, be2r = b2.reshape(1, F), g2.reshape(1, F), be2.reshape(1, F)
    b3r, g3r, be3r = b3.reshape(1, F), g3.reshape(1, F), be3.reshape(1, F)

    acc1 = _sc_agg(y1v.reshape(2 * N, HF), srcidx, dstidx,
                   zeros32).reshape(2, N, HF)
    y2v = _tc_mid(acc1, y1v, dinv, b1r, g1r, be1r, W2)
    acc2 = _sc_agg(y2v.reshape(2 * N, HF), srcidx, dstidx,
                   zeros32).reshape(2, N, HF)
    y3v = _tc_mid(acc2, y2v, dinv, b2r, g2r, be2r, W3)
    acc3 = _sc_agg(y3v.reshape(2 * N, HF), srcidx, dstidx,
                   zeros32).reshape(2, N, HF)

    batch2d = batch_idx.astype(jnp.int32).reshape(N, 1)
    return _tc_final(acc3, y3v, dinv, b3r, g3r, be3r, batch2d,
                     Wm1, bm1.reshape(1, 32), Wm2, bm2.reshape(1, 16),
                     Wm3, bm3.reshape(1, 1))


# speedup vs baseline: 17.4609x; 1.0012x over previous
import functools
import numpy as np
import jax
import jax.numpy as jnp
from jax import lax
from jax.experimental import pallas as pl
from jax.experimental.pallas import tpu as pltpu
from jax.experimental.pallas import tpu_sc as plsc

N = 50000
F = 64
HF = 32
G = 64
EPS = 1e-5
EB = 128
NBLK = 6400
EPAD = NBLK * EB
SLAB = 40
DSLAB = 40
ACC_ROWS = 50048
ZROWS = ACC_ROWS // 16
OROWS = 3128
OLAST = N - 15 * OROWS
DEGW = 16
BT = 2000
GRID = N // BT


NB = 5
D = 2


def _sc_agg_body(ytab, srcidx, dstidx, zeros_blk, out, acc, sslab, dslab,
                 rows, sems_g, sems_s):
    c = lax.axis_index("c")
    s = lax.axis_index("s")
    pltpu.sync_copy(zeros_blk, acc.at[pl.ds(s * ZROWS, ZROWS)])
    plsc.subcore_barrier()

    blocks_per_tile = NBLK // 16
    nslab = blocks_per_tile // SLAB

    def slab_body(k, _):
        row0 = s * blocks_per_tile + k * SLAB
        pltpu.sync_copy(srcidx.at[pl.ds(c * NBLK + row0, SLAB)], sslab)
        pltpu.sync_copy(dstidx.at[pl.ds(row0, SLAB)], dslab)

        for b in range(D):
            pltpu.async_copy(ytab.at[sslab.at[b]], rows[b], sems_g[b])

        def inner(o, _):
            for t in range(NB):
                i = o * NB + t
                bg = (t + D) % NB

                @pl.when(i + D >= NB)
                def _():
                    pltpu.make_async_copy(
                        rows[bg], acc.at[pl.ds(0, EB)], sems_s[bg]).wait()

                @pl.when(i + D < SLAB)
                def _():
                    pltpu.async_copy(
                        ytab.at[sslab.at[i + D]], rows[bg], sems_g[bg])

                pltpu.make_async_copy(
                    ytab.at[pl.ds(0, EB)], rows[t], sems_g[t]).wait()
                pltpu.async_copy(
                    rows[t], acc.at[dslab.at[i]], sems_s[t], add=True)
            return 0

        lax.fori_loop(0, SLAB // NB, inner, 0)
        for t in range(D, NB):
            pltpu.make_async_copy(
                rows[t], acc.at[pl.ds(0, EB)], sems_s[t]).wait()
        return 0

    lax.fori_loop(0, nslab, slab_body, 0)
    plsc.subcore_barrier()
    _writeout(acc, out, c, s)


def _writeout(acc, out, c, s):
    @pl.when(s < 15)
    def _():
        pltpu.sync_copy(
            acc.at[pl.ds(s * OROWS, OROWS)],
            out.at[pl.ds(c * N + s * OROWS, OROWS)],
        )

    @pl.when(s == 15)
    def _():
        pltpu.sync_copy(
            acc.at[pl.ds(15 * OROWS, OLAST)],
            out.at[pl.ds(c * N + 15 * OROWS, OLAST)],
        )


def _sc_deg_body(dstidx, ones_hbm, zeros_blk, out, acc, dslab, ones_v, sem):
    c = lax.axis_index("c")
    s = lax.axis_index("s")
    pltpu.sync_copy(zeros_blk, acc.at[pl.ds(s * ZROWS, ZROWS)])
    pltpu.sync_copy(ones_hbm, ones_v)
    plsc.subcore_barrier()

    w = c * 16 + s
    blocks_per_tile = NBLK // 32
    nslab = blocks_per_tile // DSLAB

    def slab_body(k, _):
        row0 = w * blocks_per_tile + k * DSLAB
        pltpu.sync_copy(dstidx.at[pl.ds(row0, DSLAB)], dslab)

        def fire(j, _):
            pltpu.async_copy(ones_v, acc.at[dslab.at[j]], sem, add=True)
            return 0

        lax.fori_loop(0, DSLAB, fire, 0)

        def drain(j, _):
            pltpu.make_async_copy(ones_v, acc.at[pl.ds(0, EB)], sem).wait()
            return 0

        lax.fori_loop(0, DSLAB, drain, 0)
        return 0

    lax.fori_loop(0, nslab, slab_body, 0)
    plsc.subcore_barrier()
    _writeout(acc, out, c, s)


@functools.cache
def _sc_kernels():
    mesh = plsc.VectorSubcoreMesh(core_axis_name="c", subcore_axis_name="s")
    agg = functools.partial(
        pl.kernel,
        out_type=jax.ShapeDtypeStruct((2 * N, HF), jnp.float32),
        mesh=mesh,
        scratch_types=[
            pltpu.VMEM_SHARED((ACC_ROWS, HF), jnp.float32),
            pltpu.VMEM((SLAB, EB), jnp.int32),
            pltpu.VMEM((SLAB, EB), jnp.int32),
            [pltpu.VMEM((EB, HF), jnp.float32) for _ in range(NB)],
            [pltpu.SemaphoreType.DMA for _ in range(NB)],
            [pltpu.SemaphoreType.DMA for _ in range(NB)],
        ],
        compiler_params=pltpu.CompilerParams(use_tc_tiling_on_sc=False),
    )(_sc_agg_body)
    deg = functools.partial(
        pl.kernel,
        out_type=jax.ShapeDtypeStruct((2 * N, DEGW), jnp.float32),
        mesh=mesh,
        scratch_types=[
            pltpu.VMEM_SHARED((ACC_ROWS, DEGW), jnp.float32),
            pltpu.VMEM((DSLAB, EB), jnp.int32),
            pltpu.VMEM((EB, DEGW), jnp.float32),
            pltpu.SemaphoreType.DMA,
        ],
        compiler_params=pltpu.CompilerParams(use_tc_tiling_on_sc=False),
    )(_sc_deg_body)
    return deg, agg


def _sc_deg(dstidx, ones_hbm, zeros_blk):
    return _sc_kernels()[0](dstidx, ones_hbm, zeros_blk)


def _sc_agg(ytab, srcidx, dstidx, zeros_blk):
    return _sc_kernels()[1](ytab, srcidx, dstidx, zeros_blk)


def _t1_body(x_ref, dp_ref, w_ref, y_ref, dinv_ref):
    deg = dp_ref[0, :, 0:1] + dp_ref[1, :, 0:1] + 1.0
    dinv = lax.rsqrt(deg)
    xb = x_ref[...]
    w = w_ref[...]
    xw = (xb[:, 0:1] * w[0:1, :] + xb[:, 1:2] * w[1:2, :]
          + xb[:, 2:3] * w[2:3, :])
    yv = xw * dinv
    y_ref[0] = yv[:, :HF]
    y_ref[1] = yv[:, HF:]
    dinv_ref[...] = dinv


def _tc_first(x, degp, w1):
    return pl.pallas_call(
        _t1_body,
        grid=(GRID,),
        in_specs=[
            pl.BlockSpec((BT, 3), lambda i: (i, 0)),
            pl.BlockSpec((2, BT, DEGW), lambda i: (0, i, 0)),
            pl.BlockSpec((3, F), lambda i: (0, 0)),
        ],
        out_specs=[
            pl.BlockSpec((2, BT, HF), lambda i: (0, i, 0)),
            pl.BlockSpec((BT, 1), lambda i: (i, 0)),
        ],
        out_shape=[
            jax.ShapeDtypeStruct((2, N, HF), jnp.float32),
            jax.ShapeDtypeStruct((N, 1), jnp.float32),
        ],
    )(x, degp, w1)


def _combine(acc_ref, y_ref, dinv_ref, b_ref, g_ref, be_ref):
    agg = jnp.concatenate(
        [acc_ref[0] + y_ref[0], acc_ref[1] + y_ref[1]], axis=1)
    pre = agg * dinv_ref[...] + b_ref[...]
    scale = g_ref[...] * lax.rsqrt(jnp.float32(1.0 + EPS))
    return jnp.maximum(pre * scale + be_ref[...], 0.0)


def _t2_body(acc_ref, y_ref, dinv_ref, b_ref, g_ref, be_ref, w_ref, yout_ref):
    h = _combine(acc_ref, y_ref, dinv_ref, b_ref, g_ref, be_ref)
    ynew = jnp.dot(h, w_ref[...], preferred_element_type=jnp.float32)
    ynew = ynew * dinv_ref[...]
    yout_ref[0] = ynew[:, :HF]
    yout_ref[1] = ynew[:, HF:]


def _tc_mid(acc, y, dinv, b, g, be, wnext):
    return pl.pallas_call(
        _t2_body,
        grid=(GRID,),
        in_specs=[
            pl.BlockSpec((2, BT, HF), lambda i: (0, i, 0)),
            pl.BlockSpec((2, BT, HF), lambda i: (0, i, 0)),
            pl.BlockSpec((BT, 1), lambda i: (i, 0)),
            pl.BlockSpec((1, F), lambda i: (0, 0)),
            pl.BlockSpec((1, F), lambda i: (0, 0)),
            pl.BlockSpec((1, F), lambda i: (0, 0)),
            pl.BlockSpec((F, F), lambda i: (0, 0)),
        ],
        out_specs=pl.BlockSpec((2, BT, HF), lambda i: (0, i, 0)),
        out_shape=jax.ShapeDtypeStruct((2, N, HF), jnp.float32),
    )(acc, y, dinv, b, g, be, wnext)


def _t4_body(acc_ref, y_ref, dinv_ref, b_ref, g_ref, be_ref, batch_ref,
             wm1_ref, bm1_ref, wm2_ref, bm2_ref, wm3_ref, bm3_ref,
             out_ref, sum_s, cnt_s, max_s):
    @pl.when(pl.program_id(0) == 0)
    def _():
        sum_s[...] = jnp.zeros_like(sum_s)
        cnt_s[...] = jnp.zeros_like(cnt_s)
        max_s[...] = jnp.full_like(max_s, -jnp.inf)

    h = _combine(acc_ref, y_ref, dinv_ref, b_ref, g_ref, be_ref)
    batch = batch_ref[...]
    oh = (batch == lax.broadcasted_iota(jnp.int32, (BT, G), 1))
    oh = oh.astype(jnp.float32)
    dn = (((0,), (0,)), ((), ()))
    sum_s[...] += lax.dot_general(oh, h, dn,
                                  preferred_element_type=jnp.float32)
    cnt_s[...] += lax.dot_general(oh, jnp.ones((BT, 1), jnp.float32), dn,
                                  preferred_element_type=jnp.float32)

    glo = jnp.min(batch)
    ghi = jnp.max(batch)
    rowid = lax.broadcasted_iota(jnp.int32, (G, F), 0)

    def mbody(g, _):
        m = jnp.max(jnp.where(batch == g, h, -jnp.inf), axis=0, keepdims=True)
        upd = jnp.where(rowid == g, jnp.broadcast_to(m, (G, F)), -jnp.inf)
        max_s[...] = jnp.maximum(max_s[...], upd)
        return 0

    lax.fori_loop(glo, ghi + 1, mbody, 0)

    @pl.when(pl.program_id(0) == GRID - 1)
    def _():
        mean = sum_s[...] / jnp.maximum(cnt_s[...], 1.0)
        xg = jnp.concatenate([mean, max_s[...]], axis=1)
        z = jnp.maximum(jnp.dot(xg, wm1_ref[...],
                                preferred_element_type=jnp.float32)
                        + bm1_ref[...], 0.0)
        z = jnp.maximum(jnp.dot(z, wm2_ref[...],
                                preferred_element_type=jnp.float32)
                        + bm2_ref[...], 0.0)
        out_ref[...] = jnp.dot(z, wm3_ref[...],
                               preferred_element_type=jnp.float32) + bm3_ref[...]


def _tc_final(acc, y, dinv, b, g, be, batch2d, wm1, bm1, wm2, bm2, wm3, bm3):
    full = lambda r, c: pl.BlockSpec((r, c), lambda i: (0, 0))
    return pl.pallas_call(
        _t4_body,
        grid=(GRID,),
        in_specs=[
            pl.BlockSpec((2, BT, HF), lambda i: (0, i, 0)),
            pl.BlockSpec((2, BT, HF), lambda i: (0, i, 0)),
            pl.BlockSpec((BT, 1), lambda i: (i, 0)),
            full(1, F), full(1, F), full(1, F),
            pl.BlockSpec((BT, 1), lambda i: (i, 0)),
            full(2 * F, 32), full(1, 32),
            full(32, 16), full(1, 16),
            full(16, 1), full(1, 1),
        ],
        out_specs=pl.BlockSpec((G, 1), lambda i: (0, 0)),
        out_shape=jax.ShapeDtypeStruct((G, 1), jnp.float32),
        scratch_shapes=[
            pltpu.VMEM((G, F), jnp.float32),
            pltpu.VMEM((G, 1), jnp.float32),
            pltpu.VMEM((G, F), jnp.float32),
        ],
    )(acc, y, dinv, b, g, be, batch2d, wm1, bm1, wm2, bm2, wm3, bm3)


def kernel(x, edge_index, batch_idx, W1, b1, g1, be1, W2, b2, g2, be2,
           W3, b3, g3, be3, Wm1, bm1, Wm2, bm2, Wm3, bm3):
    ei = edge_index.astype(jnp.int32)
    npad = EPAD - ei.shape[1]
    src = jnp.concatenate([ei[0], jnp.zeros((npad,), jnp.int32)])
    dst = jnp.concatenate([ei[1], jnp.full((npad,), N, jnp.int32)])
    srcidx = jnp.concatenate([src, src + N]).reshape(2 * NBLK, EB)
    dstidx = dst.reshape(NBLK, EB)

    ones_hbm = jnp.ones((EB, DEGW), jnp.float32)
    zeros16 = jnp.zeros((ZROWS, DEGW), jnp.float32)
    zeros32 = jnp.zeros((ZROWS, HF), jnp.float32)

    degp = _sc_deg(dstidx, ones_hbm, zeros16)
    degp = degp.reshape(2, N, DEGW)

    y1v, dinv = _tc_first(x, degp, W1)

    b1r, g1r, be1r = b1.reshape(1, F), g1.reshape(1, F), be1.reshape(1, F)
    b2r, g2r, be2r = b2.reshape(1, F), g2.reshape(1, F), be2.reshape(1, F)
    b3r, g3r, be3r = b3.reshape(1, F), g3.reshape(1, F), be3.reshape(1, F)

    acc1 = _sc_agg(y1v.reshape(2 * N, HF), srcidx, dstidx,
                   zeros32).reshape(2, N, HF)
    y2v = _tc_mid(acc1, y1v, dinv, b1r, g1r, be1r, W2)
    acc2 = _sc_agg(y2v.reshape(2 * N, HF), srcidx, dstidx,
                   zeros32).reshape(2, N, HF)
    y3v = _tc_mid(acc2, y2v, dinv, b2r, g2r, be2r, W3)
    acc3 = _sc_agg(y3v.reshape(2 * N, HF), srcidx, dstidx,
                   zeros32).reshape(2, N, HF)

    batch2d = batch_idx.astype(jnp.int32).reshape(N, 1)
    return _tc_final(acc3, y3v, dinv, b3r, g3r, be3r, batch2d,
                     Wm1, bm1.reshape(1, 32), Wm2, bm2.reshape(1, 16),
                     Wm3, bm3.reshape(1, 1))
